# baseline (speedup 1.0000x reference)
import jax
import jax.numpy as jnp
from jax.experimental import pallas as pl
from jax.experimental.pallas import tpu as pltpu

NBITS = 8192
KWIN = 10
BLK = 256
NBLKS = NBITS // BLK
KSPLIT = 2
KB = NBITS // KSPLIT


def _fc_body(x_ref, w_ref, b_ref, o_ref, acc_ref, cand_ref):
    i = pl.program_id(0)
    j = pl.program_id(1)
    part = jax.lax.dot_general(
        x_ref[...], w_ref[...],
        dimension_numbers=(((1,), (1,)), ((), ())),
        preferred_element_type=jnp.float32,
    )

    @pl.when(j == 0)
    def _():
        acc_ref[:, pl.ds(i * BLK, BLK)] = part + b_ref[...]

    @pl.when(j == KSPLIT - 1)
    def _():
        v = acc_ref[:, pl.ds(i * BLK, BLK)] + part
        acc_ref[:, pl.ds(i * BLK, BLK)] = v

        @pl.when(i == 0)
        def _():
            cand_ref[...] = jnp.full((1, KWIN * BLK), -jnp.inf, jnp.float32)

        vv = v
        for t in range(KWIN):
            c = cand_ref[:, t * BLK:(t + 1) * BLK]
            hi = jnp.maximum(c, vv)
            vv = jnp.minimum(c, vv)
            cand_ref[:, t * BLK:(t + 1) * BLK] = hi

        @pl.when(i == NBLKS - 1)
        def _():
            cand = cand_ref[...]

            def step(_, carry):
                thr, cnt = carry
                masked = jnp.where(cand < thr, cand, -jnp.inf)
                m = jnp.max(masked)
                c2 = jnp.sum((cand == m).astype(jnp.int32))
                take = cnt < KWIN
                return jnp.where(take, m, thr), jnp.where(take, cnt + c2, cnt)

            thr, _ = jax.lax.fori_loop(
                0, KWIN, step, (jnp.float32(jnp.inf), jnp.int32(0))
            )
            out = acc_ref[...]
            o_ref[...] = jnp.where(out >= thr, out, 0.0)


def kernel(x, W, b):
    b_row = b.reshape(1, NBITS)
    return pl.pallas_call(
        _fc_body,
        grid=(NBLKS, KSPLIT),
        in_specs=[
            pl.BlockSpec((1, KB), lambda i, j: (0, j)),
            pl.BlockSpec((BLK, KB), lambda i, j: (i, j)),
            pl.BlockSpec((1, BLK), lambda i, j: (0, i)),
        ],
        out_specs=pl.BlockSpec((1, NBITS), lambda i, j: (0, 0)),
        out_shape=jax.ShapeDtypeStruct((1, NBITS), jnp.float32),
        scratch_shapes=[
            pltpu.VMEM((1, NBITS), jnp.float32),
            pltpu.VMEM((1, KWIN * BLK), jnp.float32),
        ],
    )(x, W, b_row)


# speedup vs baseline: 1.2050x; 1.2050x over previous
import jax
import jax.numpy as jnp
from jax.experimental import pallas as pl
from jax.experimental.pallas import tpu as pltpu

NBITS = 8192
KWIN = 10
BLK = 256
NBLKS = NBITS // BLK


def _fc_body(x_ref, w_ref, b_ref, o_ref, acc_ref, cand_ref):
    i = pl.program_id(0)
    part = jax.lax.dot_general(
        x_ref[...], w_ref[...],
        dimension_numbers=(((1,), (1,)), ((), ())),
        preferred_element_type=jnp.float32,
    ) + b_ref[...]
    acc_ref[:, pl.ds(i * BLK, BLK)] = part

    @pl.when(i == 0)
    def _():
        cand_ref[...] = jnp.full((KWIN, BLK), -jnp.inf, jnp.float32)

    v = part
    for t in range(KWIN):
        c = cand_ref[pl.ds(t, 1), :]
        hi = jnp.maximum(c, v)
        v = jnp.minimum(c, v)
        cand_ref[pl.ds(t, 1), :] = hi

    @pl.when(i == NBLKS - 1)
    def _():
        cand = cand_ref[...]

        vals = []
        m = jnp.float32(jnp.inf)
        for _ in range(KWIN):
            m = jnp.max(jnp.where(cand < m, cand, -jnp.inf))
            vals.append(m)
        cnts = [jnp.sum((cand == v).astype(jnp.int32)) for v in vals]
        thr = vals[0]
        cum = cnts[0]
        for r in range(1, KWIN):
            need = cum < KWIN
            thr = jnp.where(need, vals[r], thr)
            cum = jnp.where(need, cum + cnts[r], cum)

        out = acc_ref[...]
        o_ref[...] = jnp.where(out >= thr, out, 0.0)


def kernel(x, W, b):
    b_row = b.reshape(1, NBITS)
    return pl.pallas_call(
        _fc_body,
        grid=(NBLKS,),
        in_specs=[
            pl.BlockSpec((1, NBITS), lambda i: (0, 0)),
            pl.BlockSpec((BLK, NBITS), lambda i: (i, 0)),
            pl.BlockSpec((1, BLK), lambda i: (0, i)),
        ],
        out_specs=pl.BlockSpec((1, NBITS), lambda i: (0, 0)),
        out_shape=jax.ShapeDtypeStruct((1, NBITS), jnp.float32),
        scratch_shapes=[
            pltpu.VMEM((1, NBITS), jnp.float32),
            pltpu.VMEM((KWIN, BLK), jnp.float32),
        ],
    )(x, W, b_row)
